# baseline (speedup 1.0000x reference)
import jax
import jax.numpy as jnp
from jax import lax
from jax.experimental import pallas as pl
from jax.experimental.pallas import tpu as pltpu
from jax.experimental.pallas import tpu_sc as plsc

NUM_USERS = 5000
NUM_ITEMS = 5000
N_NODES = NUM_USERS + NUM_ITEMS
D = 128
E = 320000
B = 4096

NW = 32
EPW = E // NW
K = 80
NCHUNK = EPW // K
NBUF = 4
ACC_ROWS = 10112
ROWS_PER_TILE = ACC_ROWS // 16

_MESH = plsc.VectorSubcoreMesh(core_axis_name="c", subcore_axis_name="s")


def _splat_lane(v16, j):
    return lax.broadcast_in_dim(lax.slice(v16, (j,), (j + 1,)), (16,), (0,))


def _spmm_body(src_hbm, cols_hbm, rows_hbm, vals_hbm, zeros_hbm, out_hbm,
               colsr, rowsr, valsr, bufs, acc_sh, isem, gsem, ssem):
    c = lax.axis_index("c")
    s = lax.axis_index("s")
    w = c * 16 + s
    pltpu.sync_copy(zeros_hbm, acc_sh.at[pl.ds(s * ROWS_PER_TILE, ROWS_PER_TILE)])

    def issue_stage(ci, p):
        sl = pl.ds(w * EPW + ci * K, K)
        pltpu.async_copy(cols_hbm.at[sl], colsr.at[p], isem.at[p])
        pltpu.async_copy(rows_hbm.at[sl], rowsr.at[p], isem.at[p])
        pltpu.async_copy(vals_hbm.at[sl], valsr.at[p], isem.at[p])

    def wait_stage(p):
        sl = pl.ds(0, K)
        pltpu.make_async_copy(cols_hbm.at[sl], colsr.at[p], isem.at[p]).wait()
        pltpu.make_async_copy(rows_hbm.at[sl], rowsr.at[p], isem.at[p]).wait()
        pltpu.make_async_copy(vals_hbm.at[sl], valsr.at[p], isem.at[p]).wait()

    def issue_gather(p):
        pltpu.async_copy(src_hbm.at[colsr.at[p]], bufs.at[p], gsem.at[p])

    def wait_gather(p):
        pltpu.make_async_copy(
            src_hbm.at[pl.ds(0, K)], bufs.at[p], gsem.at[p]).wait()

    def issue_scatter(p):
        pltpu.async_copy(bufs.at[p], acc_sh.at[rowsr.at[p]], ssem.at[p],
                         add=True)

    def wait_scatter(p):
        pltpu.make_async_copy(
            bufs.at[p], acc_sh.at[pl.ds(0, K)], ssem.at[p]).wait()

    def scale(p):
        def grp(j, carry):
            v16 = valsr[p, pl.ds(j * 16, 16)]
            for jj in range(16):
                sp = _splat_lane(v16, jj)
                row = j * 16 + jj
                for kk in range(D // 16):
                    sl = pl.ds(kk * 16, 16)
                    bufs[p, row, sl] = bufs[p, row, sl] * sp
            return carry

        lax.fori_loop(0, K // 16, grp, 0)

    plsc.subcore_barrier()

    def body(ci, q2, q1, p, first=False, stage=True, gather=True):
        if not first:
            wait_scatter(q2)
        if stage:
            issue_stage(ci + 2, q2)
        if gather:
            wait_stage(q1)
            issue_gather(q1)
        wait_gather(p)
        scale(p)
        issue_scatter(p)

    issue_stage(0, 0)
    issue_stage(1, 1)
    wait_stage(0)
    issue_gather(0)
    body(0, 2, 1, 0, first=True)
    body(1, 3, 2, 1, first=True)

    def quad(i, carry):
        for q in range(4):
            ci = 2 + 4 * i + q
            body(ci, q, (q + 3) % NBUF, (q + 2) % NBUF)
        return carry

    lax.fori_loop(0, (NCHUNK - 5) // 4, quad, 0)

    body(122, 0, 3, 2)
    body(123, 1, 0, 3, stage=False)
    body(124, 2, 0, 0, stage=False, gather=False)
    wait_scatter(3)
    wait_scatter(0)

    plsc.subcore_barrier()
    sl = pl.ds(s * ROWS_PER_TILE, ROWS_PER_TILE)
    pltpu.sync_copy(acc_sh.at[sl], out_hbm.at[c].at[sl])


_spmm_call = pl.kernel(
    _spmm_body,
    out_type=jax.ShapeDtypeStruct((2, ACC_ROWS, D), jnp.float32),
    mesh=_MESH,
    scratch_types=[
        pltpu.VMEM((NBUF, K), jnp.int32),
        pltpu.VMEM((NBUF, K), jnp.int32),
        pltpu.VMEM((NBUF, K), jnp.float32),
        pltpu.VMEM((NBUF, K, D), jnp.float32),
        pltpu.VMEM_SHARED((ACC_ROWS, D), jnp.float32),
        pltpu.SemaphoreType.DMA((NBUF,)),
        pltpu.SemaphoreType.DMA((NBUF,)),
        pltpu.SemaphoreType.DMA((NBUF,)),
    ],
)


def _gather_body(mn_lo, mn_hi, msub1, msvd_lo, msvd_hi, idx7_hbm, out_hbm,
                 idx_v, buf, sem):
    c = lax.axis_index("c")
    s = lax.axis_index("s")
    w = c * 16 + s
    bpw = B // NW
    tables = [mn_lo, mn_hi, mn_hi, msub1, msub1, msvd_lo, msvd_hi]
    for t, tab in enumerate(tables):
        pltpu.sync_copy(idx7_hbm.at[t].at[w], idx_v)
        pltpu.async_copy(tab.at[idx_v], buf, sem).wait()
        pltpu.sync_copy(buf, out_hbm.at[t].at[pl.ds(w * bpw, bpw)])


_gather_call = pl.kernel(
    _gather_body,
    out_type=jax.ShapeDtypeStruct((7, B, D), jnp.float32),
    mesh=_MESH,
    scratch_types=[
        pltpu.VMEM((B // NW,), jnp.int32),
        pltpu.VMEM((B // NW, D), jnp.float32),
        pltpu.SemaphoreType.DMA,
    ],
)


def _combine_body(p_ref, o_ref):
    o_ref[...] = p_ref[0] + p_ref[1]


def _combine(partials):
    return pl.pallas_call(
        _combine_body,
        grid=(5,),
        in_specs=[pl.BlockSpec((2, 2000, D), lambda i: (0, i, 0))],
        out_specs=pl.BlockSpec((2000, D), lambda i: (i, 0)),
        out_shape=jax.ShapeDtypeStruct((N_NODES, D), jnp.float32),
    )(partials)


def _normalize(x):
    nrm = jnp.sqrt(jnp.sum(x * x, axis=1, keepdims=True))
    return x / jnp.maximum(nrm, 1e-12)


def _prep_body(ego, n1, n2, pn3, s1, s2, ps3, v1, v2, pv3,
               mn_ref, msub_ref, msvd_ref):
    mn_ref[...] = (ego[...] + n1[...] + n2[...] + pn3[0] + pn3[1]) * 0.25
    msub_ref[...] = (ego[...] + s1[...] + s2[...] + ps3[0] + ps3[1]) * 0.25
    msvd_ref[...] = _normalize(
        (ego[...] + v1[...] + v2[...] + pv3[0] + pv3[1]) * 0.25)


def _prep(ego, n1, n2, pn3, s1, s2, ps3, v1, v2, pv3):
    full = pl.BlockSpec((2000, D), lambda i: (i, 0))
    part = pl.BlockSpec((2, 2000, D), lambda i: (0, i, 0))
    return pl.pallas_call(
        _prep_body,
        grid=(5,),
        in_specs=[full, full, full, part, full, full, part, full, full, part],
        out_specs=[full, full, full],
        out_shape=[jax.ShapeDtypeStruct((N_NODES, D), jnp.float32)] * 3,
    )(ego, n1, n2, pn3, s1, s2, ps3, v1, v2, pv3)


def _b1_body(g7, u1n_ref, i1n_ref, sup_ref, pos_ref):
    u = g7[0]
    i = g7[1]
    ni = g7[2]
    u1n = _normalize(g7[3])
    i1n = _normalize(g7[4])
    us = g7[5]
    iss = g7[6]
    u1n_ref[...] = u1n
    i1n_ref[...] = i1n
    sup_ref[...] = (jnp.sum(u * i, axis=1) - jnp.sum(u * ni, axis=1))[:, None]
    pos_u = jnp.sum(u1n * u, axis=1)
    pos_i = jnp.sum(i1n * i, axis=1)
    spos_u = jnp.sum(u * us, axis=1)
    spos_i = jnp.sum(i * iss, axis=1)
    pos_ref[...] = jnp.stack([pos_u, pos_i, spos_u, spos_i], axis=1)


def _b1(g7):
    return pl.pallas_call(
        _b1_body,
        out_shape=[
            jax.ShapeDtypeStruct((B, D), jnp.float32),
            jax.ShapeDtypeStruct((B, D), jnp.float32),
            jax.ShapeDtypeStruct((B, 1), jnp.float32),
            jax.ShapeDtypeStruct((B, 4), jnp.float32),
        ],
    )(g7)


def _mm_body(a_ref, b_ref, pos_ref, o_ref):
    o_ref[0, :, :NUM_USERS] = lax.dot_general(
        a_ref[0], b_ref[0], (((1,), (0,)), ((), ())),
        preferred_element_type=jnp.float32) - pos_ref[0]


def _mm(a_stack, bt_stack, pos3):
    mt = 8
    mblk = B // mt
    return pl.pallas_call(
        _mm_body,
        grid=(4, mt),
        in_specs=[
            pl.BlockSpec((1, mblk, D), lambda a, m: (a, m, 0)),
            pl.BlockSpec((1, D, NUM_USERS), lambda a, m: (a, 0, 0)),
            pl.BlockSpec((1, mblk, 1), lambda a, m: (a, m, 0)),
        ],
        out_specs=pl.BlockSpec((1, mblk, 5120), lambda a, m: (a, m, 0)),
        out_shape=jax.ShapeDtypeStruct((4, B, 5120), jnp.float32),
    )(a_stack, bt_stack, pos3)


def _edge_args(idx, val):
    return idx[1], idx[0], val


def _chain(ego, idx, val, zeros):
    cols, rows, vals = _edge_args(idx, val)
    p1 = _spmm_call(ego, cols, rows, vals, zeros)
    x1 = _combine(p1)
    p2 = _spmm_call(x1, cols, rows, vals, zeros)
    x2 = _combine(p2)
    p3 = _spmm_call(x2, cols, rows, vals, zeros)
    return x1, x2, p3


def kernel(user_embeddings, item_embeddings, norm_adj_idx, norm_adj_val,
           svd_adj_idx, svd_adj_val, sub_graph1_idx, sub_graph1_val,
           sub_graph2_idx, sub_graph2_val, users, items, neg_items):
    ego = jnp.concatenate([user_embeddings, item_embeddings], axis=0)
    zeros = jnp.zeros((ROWS_PER_TILE, D), jnp.float32)

    n1, n2, pn3 = _chain(ego, norm_adj_idx, norm_adj_val, zeros)
    s1, s2, ps3 = _chain(ego, sub_graph1_idx, sub_graph1_val, zeros)
    v1, v2, pv3 = _chain(ego, svd_adj_idx, svd_adj_val, zeros)

    mn, msub1, msvd = _prep(ego, n1, n2, pn3, s1, s2, ps3, v1, v2, pv3)
    mn_lo, mn_hi = mn[:NUM_USERS], mn[NUM_USERS:]
    msvd_lo, msvd_hi = msvd[:NUM_USERS], msvd[NUM_USERS:]

    idx7 = jnp.stack([
        users, items, neg_items, users, items + NUM_USERS, users, items,
    ]).reshape(7, NW, B // NW).astype(jnp.int32)
    g7 = _gather_call(mn_lo, mn_hi, msub1, msvd_lo, msvd_hi, idx7)

    u1n, i1n, sup_col, pos_cols = _b1(g7)
    a_stack = jnp.stack([u1n, i1n, g7[0], g7[1]]).astype(jnp.bfloat16)
    bt_stack = jnp.stack(
        [mn_lo, mn_hi, msvd_lo, msvd_hi]).astype(jnp.bfloat16).transpose(0, 2, 1)
    pos3 = pos_cols.T.reshape(4, B, 1)
    out4 = _mm(a_stack, bt_stack, pos3)

    return (sup_col[:, 0], out4[0, :, :NUM_USERS], out4[1, :, :NUM_USERS], out4[2, :, :NUM_USERS], out4[3, :, :NUM_USERS])


# speedup vs baseline: 1.0037x; 1.0037x over previous
import jax
import jax.numpy as jnp
from jax import lax
from jax.experimental import pallas as pl
from jax.experimental.pallas import tpu as pltpu
from jax.experimental.pallas import tpu_sc as plsc

NUM_USERS = 5000
NUM_ITEMS = 5000
N_NODES = NUM_USERS + NUM_ITEMS
D = 128
E = 320000
B = 4096

NW = 32
EPW = E // NW
K = 80
NCHUNK = EPW // K
NBUF = 4
ACC_ROWS = 10112
ROWS_PER_TILE = ACC_ROWS // 16

_MESH = plsc.VectorSubcoreMesh(core_axis_name="c", subcore_axis_name="s")


def _splat_lane(v16, j):
    return lax.broadcast_in_dim(lax.slice(v16, (j,), (j + 1,)), (16,), (0,))


def _spmm_body(src_hbm, edges_hbm, zeros_hbm, out_hbm,
               edger, bufs, acc_sh, isem, gsem, ssem):
    c = lax.axis_index("c")
    s = lax.axis_index("s")
    w = c * 16 + s
    pltpu.sync_copy(zeros_hbm, acc_sh.at[pl.ds(s * ROWS_PER_TILE, ROWS_PER_TILE)])

    def issue_stage(ci, p):
        pltpu.async_copy(edges_hbm.at[w * NCHUNK + ci],
                         edger.at[pl.ds(3 * p, 3)], isem.at[p])

    def wait_stage(p):
        pltpu.make_async_copy(
            edges_hbm.at[0], edger.at[pl.ds(3 * p, 3)], isem.at[p]).wait()

    def issue_gather(p):
        pltpu.async_copy(
            src_hbm.at[edger.at[3 * p]], bufs.at[p], gsem.at[p])

    def wait_gather(p):
        pltpu.make_async_copy(
            src_hbm.at[pl.ds(0, K)], bufs.at[p], gsem.at[p]).wait()

    def issue_scatter(p):
        pltpu.async_copy(bufs.at[p], acc_sh.at[edger.at[3 * p + 1]], ssem.at[p],
                         add=True)

    def wait_scatter(p):
        pltpu.make_async_copy(
            bufs.at[p], acc_sh.at[pl.ds(0, K)], ssem.at[p]).wait()

    def scale(p):
        def grp(j, carry):
            v16 = lax.bitcast_convert_type(edger[3 * p + 2, pl.ds(j * 16, 16)], jnp.float32)
            for jj in range(16):
                sp = _splat_lane(v16, jj)
                row = j * 16 + jj
                for kk in range(D // 16):
                    sl = pl.ds(kk * 16, 16)
                    bufs[p, row, sl] = bufs[p, row, sl] * sp
            return carry

        lax.fori_loop(0, K // 16, grp, 0)

    plsc.subcore_barrier()

    def body(ci, q2, q1, p, first=False, stage=True, gather=True):
        if not first:
            wait_scatter(q2)
        if stage:
            issue_stage(ci + 2, q2)
        if gather:
            wait_stage(q1)
            issue_gather(q1)
        wait_gather(p)
        scale(p)
        issue_scatter(p)

    issue_stage(0, 0)
    issue_stage(1, 1)
    wait_stage(0)
    issue_gather(0)
    body(0, 2, 1, 0, first=True)
    body(1, 3, 2, 1, first=True)

    def quad(i, carry):
        for q in range(4):
            ci = 2 + 4 * i + q
            body(ci, q, (q + 3) % NBUF, (q + 2) % NBUF)
        return carry

    lax.fori_loop(0, (NCHUNK - 5) // 4, quad, 0)

    body(122, 0, 3, 2)
    body(123, 1, 0, 3, stage=False)
    body(124, 2, 0, 0, stage=False, gather=False)
    wait_scatter(3)
    wait_scatter(0)

    plsc.subcore_barrier()
    sl = pl.ds(s * ROWS_PER_TILE, ROWS_PER_TILE)
    pltpu.sync_copy(acc_sh.at[sl], out_hbm.at[c].at[sl])


_spmm_call = pl.kernel(
    _spmm_body,
    out_type=jax.ShapeDtypeStruct((2, ACC_ROWS, D), jnp.float32),
    mesh=_MESH,
    scratch_types=[
        pltpu.VMEM((3 * NBUF, K), jnp.int32),
        pltpu.VMEM((NBUF, K, D), jnp.float32),
        pltpu.VMEM_SHARED((ACC_ROWS, D), jnp.float32),
        pltpu.SemaphoreType.DMA((NBUF,)),
        pltpu.SemaphoreType.DMA((NBUF,)),
        pltpu.SemaphoreType.DMA((NBUF,)),
    ],
)


def _gather_body(mn_lo, mn_hi, msub1, msvd_lo, msvd_hi, idx7_hbm, out_hbm,
                 idx_v, buf, sem):
    c = lax.axis_index("c")
    s = lax.axis_index("s")
    w = c * 16 + s
    bpw = B // NW
    tables = [mn_lo, mn_hi, mn_hi, msub1, msub1, msvd_lo, msvd_hi]
    for t, tab in enumerate(tables):
        pltpu.sync_copy(idx7_hbm.at[t].at[w], idx_v)
        pltpu.async_copy(tab.at[idx_v], buf, sem).wait()
        pltpu.sync_copy(buf, out_hbm.at[t].at[pl.ds(w * bpw, bpw)])


_gather_call = pl.kernel(
    _gather_body,
    out_type=jax.ShapeDtypeStruct((7, B, D), jnp.float32),
    mesh=_MESH,
    scratch_types=[
        pltpu.VMEM((B // NW,), jnp.int32),
        pltpu.VMEM((B // NW, D), jnp.float32),
        pltpu.SemaphoreType.DMA,
    ],
)


def _combine_body(p_ref, o_ref):
    o_ref[...] = p_ref[0] + p_ref[1]


def _combine(partials):
    return pl.pallas_call(
        _combine_body,
        grid=(5,),
        in_specs=[pl.BlockSpec((2, 2000, D), lambda i: (0, i, 0))],
        out_specs=pl.BlockSpec((2000, D), lambda i: (i, 0)),
        out_shape=jax.ShapeDtypeStruct((N_NODES, D), jnp.float32),
    )(partials)


def _normalize(x):
    nrm = jnp.sqrt(jnp.sum(x * x, axis=1, keepdims=True))
    return x / jnp.maximum(nrm, 1e-12)


def _prep_body(ego, n1, n2, pn3, s1, s2, ps3, v1, v2, pv3,
               mn_ref, msub_ref, msvd_ref):
    mn_ref[...] = (ego[...] + n1[...] + n2[...] + pn3[0] + pn3[1]) * 0.25
    msub_ref[...] = (ego[...] + s1[...] + s2[...] + ps3[0] + ps3[1]) * 0.25
    msvd_ref[...] = _normalize(
        (ego[...] + v1[...] + v2[...] + pv3[0] + pv3[1]) * 0.25)


def _prep(ego, n1, n2, pn3, s1, s2, ps3, v1, v2, pv3):
    full = pl.BlockSpec((2000, D), lambda i: (i, 0))
    part = pl.BlockSpec((2, 2000, D), lambda i: (0, i, 0))
    return pl.pallas_call(
        _prep_body,
        grid=(5,),
        in_specs=[full, full, full, part, full, full, part, full, full, part],
        out_specs=[full, full, full],
        out_shape=[jax.ShapeDtypeStruct((N_NODES, D), jnp.float32)] * 3,
    )(ego, n1, n2, pn3, s1, s2, ps3, v1, v2, pv3)


def _b1_body(g7, u1n_ref, i1n_ref, sup_ref, pos_ref):
    u = g7[0]
    i = g7[1]
    ni = g7[2]
    u1n = _normalize(g7[3])
    i1n = _normalize(g7[4])
    us = g7[5]
    iss = g7[6]
    u1n_ref[...] = u1n
    i1n_ref[...] = i1n
    sup_ref[...] = (jnp.sum(u * i, axis=1) - jnp.sum(u * ni, axis=1))[:, None]
    pos_u = jnp.sum(u1n * u, axis=1)
    pos_i = jnp.sum(i1n * i, axis=1)
    spos_u = jnp.sum(u * us, axis=1)
    spos_i = jnp.sum(i * iss, axis=1)
    pos_ref[...] = jnp.stack([pos_u, pos_i, spos_u, spos_i], axis=1)


def _b1(g7):
    return pl.pallas_call(
        _b1_body,
        out_shape=[
            jax.ShapeDtypeStruct((B, D), jnp.float32),
            jax.ShapeDtypeStruct((B, D), jnp.float32),
            jax.ShapeDtypeStruct((B, 1), jnp.float32),
            jax.ShapeDtypeStruct((B, 4), jnp.float32),
        ],
    )(g7)


def _mm_body(a_ref, b_ref, pos_ref, o_ref):
    o_ref[0] = lax.dot_general(
        a_ref[0], b_ref[0], (((1,), (1,)), ((), ())),
        preferred_element_type=jnp.float32) - pos_ref[0]


def _mm(a_stack, b_stack, pos3):
    mt = 4
    mblk = B // mt
    return pl.pallas_call(
        _mm_body,
        grid=(4, mt),
        in_specs=[
            pl.BlockSpec((1, mblk, D), lambda a, m: (a, m, 0)),
            pl.BlockSpec((1, NUM_USERS, D), lambda a, m: (a, 0, 0)),
            pl.BlockSpec((1, mblk, 1), lambda a, m: (a, m, 0)),
        ],
        out_specs=pl.BlockSpec((1, mblk, NUM_USERS), lambda a, m: (a, m, 0)),
        out_shape=jax.ShapeDtypeStruct((4, B, NUM_USERS), jnp.float32),
    )(a_stack, b_stack, pos3)


def _edge_pack(idx, val):
    cols = idx[1].reshape(NW, NCHUNK, 1, K)
    rows = idx[0].reshape(NW, NCHUNK, 1, K)
    vals = lax.bitcast_convert_type(val, jnp.int32).reshape(NW, NCHUNK, 1, K)
    return jnp.concatenate([cols, rows, vals], axis=2).reshape(
        NW * NCHUNK, 3, K)


def _chain(ego, idx, val, zeros):
    edges = _edge_pack(idx, val)
    p1 = _spmm_call(ego, edges, zeros)
    x1 = _combine(p1)
    p2 = _spmm_call(x1, edges, zeros)
    x2 = _combine(p2)
    p3 = _spmm_call(x2, edges, zeros)
    return x1, x2, p3


def kernel(user_embeddings, item_embeddings, norm_adj_idx, norm_adj_val,
           svd_adj_idx, svd_adj_val, sub_graph1_idx, sub_graph1_val,
           sub_graph2_idx, sub_graph2_val, users, items, neg_items):
    ego = jnp.concatenate([user_embeddings, item_embeddings], axis=0)
    zeros = jnp.zeros((ROWS_PER_TILE, D), jnp.float32)

    n1, n2, pn3 = _chain(ego, norm_adj_idx, norm_adj_val, zeros)
    s1, s2, ps3 = _chain(ego, sub_graph1_idx, sub_graph1_val, zeros)
    v1, v2, pv3 = _chain(ego, svd_adj_idx, svd_adj_val, zeros)

    mn, msub1, msvd = _prep(ego, n1, n2, pn3, s1, s2, ps3, v1, v2, pv3)
    mn_lo, mn_hi = mn[:NUM_USERS], mn[NUM_USERS:]
    msvd_lo, msvd_hi = msvd[:NUM_USERS], msvd[NUM_USERS:]

    idx7 = jnp.stack([
        users, items, neg_items, users, items + NUM_USERS, users, items,
    ]).reshape(7, NW, B // NW).astype(jnp.int32)
    g7 = _gather_call(mn_lo, mn_hi, msub1, msvd_lo, msvd_hi, idx7)

    u1n, i1n, sup_col, pos_cols = _b1(g7)
    a_stack = jnp.stack([u1n, i1n, g7[0], g7[1]])
    b_stack = jnp.stack([mn_lo, mn_hi, msvd_lo, msvd_hi])
    pos3 = pos_cols.T.reshape(4, B, 1)
    out4 = _mm(a_stack, b_stack, pos3)

    return (sup_col[:, 0], out4[0], out4[1], out4[2], out4[3])


# speedup vs baseline: 1.0143x; 1.0106x over previous
import jax
import jax.numpy as jnp
from jax import lax
from jax.experimental import pallas as pl
from jax.experimental.pallas import tpu as pltpu
from jax.experimental.pallas import tpu_sc as plsc

NUM_USERS = 5000
NUM_ITEMS = 5000
N_NODES = NUM_USERS + NUM_ITEMS
D = 128
E = 320000
B = 4096

NW = 32
EPW = E // NW
K = 80
NCHUNK = EPW // K
NBUF = 4
ACC_ROWS = 10112
ROWS_PER_TILE = ACC_ROWS // 16

_MESH = plsc.VectorSubcoreMesh(core_axis_name="c", subcore_axis_name="s")


def _splat_lane(v16, j):
    return lax.broadcast_in_dim(lax.slice(v16, (j,), (j + 1,)), (16,), (0,))


def _spmm_body(src_hbm, cols_hbm, rows_hbm, vals_hbm, zeros_hbm, out_hbm,
               colsr, rowsr, valsr, bufs, acc_sh, isem, gsem, ssem):
    c = lax.axis_index("c")
    s = lax.axis_index("s")
    w = c * 16 + s
    pltpu.sync_copy(zeros_hbm, acc_sh.at[pl.ds(s * ROWS_PER_TILE, ROWS_PER_TILE)])

    def issue_stage(ci, p):
        sl = pl.ds(w * EPW + ci * K, K)
        pltpu.async_copy(cols_hbm.at[sl], colsr.at[p], isem.at[p])
        pltpu.async_copy(rows_hbm.at[sl], rowsr.at[p], isem.at[p])
        pltpu.async_copy(vals_hbm.at[sl], valsr.at[p], isem.at[p])

    def wait_stage(p):
        sl = pl.ds(0, K)
        pltpu.make_async_copy(cols_hbm.at[sl], colsr.at[p], isem.at[p]).wait()
        pltpu.make_async_copy(rows_hbm.at[sl], rowsr.at[p], isem.at[p]).wait()
        pltpu.make_async_copy(vals_hbm.at[sl], valsr.at[p], isem.at[p]).wait()

    def issue_gather(p):
        pltpu.async_copy(src_hbm.at[colsr.at[p]], bufs.at[p], gsem.at[p])

    def wait_gather(p):
        pltpu.make_async_copy(
            src_hbm.at[pl.ds(0, K)], bufs.at[p], gsem.at[p]).wait()

    def issue_scatter(p):
        pltpu.async_copy(bufs.at[p], acc_sh.at[rowsr.at[p]], ssem.at[p],
                         add=True)

    def wait_scatter(p):
        pltpu.make_async_copy(
            bufs.at[p], acc_sh.at[pl.ds(0, K)], ssem.at[p]).wait()

    def scale(p):
        def grp(j, carry):
            v16 = valsr[p, pl.ds(j * 16, 16)]
            for jj in range(16):
                sp = _splat_lane(v16, jj)
                row = j * 16 + jj
                for kk in range(D // 16):
                    sl = pl.ds(kk * 16, 16)
                    bufs[p, row, sl] = bufs[p, row, sl] * sp
            return carry

        lax.fori_loop(0, K // 16, grp, 0)

    plsc.subcore_barrier()

    def body(ci, q2, q1, p, first=False, stage=True, gather=True):
        if gather:
            wait_stage(q1)
            issue_gather(q1)
        if not first:
            wait_scatter(q2)
        if stage:
            issue_stage(ci + 2, q2)
        wait_gather(p)
        scale(p)
        issue_scatter(p)

    issue_stage(0, 0)
    issue_stage(1, 1)
    wait_stage(0)
    issue_gather(0)
    body(0, 2, 1, 0, first=True)
    body(1, 3, 2, 1, first=True)

    def quad(i, carry):
        for q in range(4):
            ci = 2 + 4 * i + q
            body(ci, q, (q + 3) % NBUF, (q + 2) % NBUF)
        return carry

    lax.fori_loop(0, (NCHUNK - 5) // 4, quad, 0)

    body(122, 0, 3, 2)
    body(123, 1, 0, 3, stage=False)
    body(124, 2, 0, 0, stage=False, gather=False)
    wait_scatter(3)
    wait_scatter(0)

    plsc.subcore_barrier()
    sl = pl.ds(s * ROWS_PER_TILE, ROWS_PER_TILE)
    pltpu.sync_copy(acc_sh.at[sl], out_hbm.at[c].at[sl])


_spmm_call = pl.kernel(
    _spmm_body,
    out_type=jax.ShapeDtypeStruct((2, ACC_ROWS, D), jnp.float32),
    mesh=_MESH,
    scratch_types=[
        pltpu.VMEM((NBUF, K), jnp.int32),
        pltpu.VMEM((NBUF, K), jnp.int32),
        pltpu.VMEM((NBUF, K), jnp.float32),
        pltpu.VMEM((NBUF, K, D), jnp.float32),
        pltpu.VMEM_SHARED((ACC_ROWS, D), jnp.float32),
        pltpu.SemaphoreType.DMA((NBUF,)),
        pltpu.SemaphoreType.DMA((NBUF,)),
        pltpu.SemaphoreType.DMA((NBUF,)),
    ],
)


def _gather_body(mn_lo, mn_hi, msub1, msvd_lo, msvd_hi, idx7_hbm, out_hbm,
                 idx_v, buf, sem):
    c = lax.axis_index("c")
    s = lax.axis_index("s")
    w = c * 16 + s
    bpw = B // NW
    tables = [mn_lo, mn_hi, mn_hi, msub1, msub1, msvd_lo, msvd_hi]
    for t, tab in enumerate(tables):
        pltpu.sync_copy(idx7_hbm.at[t].at[w], idx_v)
        pltpu.async_copy(tab.at[idx_v], buf, sem).wait()
        pltpu.sync_copy(buf, out_hbm.at[t].at[pl.ds(w * bpw, bpw)])


_gather_call = pl.kernel(
    _gather_body,
    out_type=jax.ShapeDtypeStruct((7, B, D), jnp.float32),
    mesh=_MESH,
    scratch_types=[
        pltpu.VMEM((B // NW,), jnp.int32),
        pltpu.VMEM((B // NW, D), jnp.float32),
        pltpu.SemaphoreType.DMA,
    ],
)


def _combine_body(p_ref, o_ref):
    o_ref[...] = p_ref[0] + p_ref[1]


def _combine(partials):
    return pl.pallas_call(
        _combine_body,
        grid=(5,),
        in_specs=[pl.BlockSpec((2, 2000, D), lambda i: (0, i, 0))],
        out_specs=pl.BlockSpec((2000, D), lambda i: (i, 0)),
        out_shape=jax.ShapeDtypeStruct((N_NODES, D), jnp.float32),
    )(partials)


def _normalize(x):
    nrm = jnp.sqrt(jnp.sum(x * x, axis=1, keepdims=True))
    return x / jnp.maximum(nrm, 1e-12)


def _prep_body(ego, n1, n2, pn3, s1, s2, ps3, v1, v2, pv3,
               mn_ref, msub_ref, msvd_ref):
    mn_ref[...] = (ego[...] + n1[...] + n2[...] + pn3[0] + pn3[1]) * 0.25
    msub_ref[...] = (ego[...] + s1[...] + s2[...] + ps3[0] + ps3[1]) * 0.25
    msvd_ref[...] = _normalize(
        (ego[...] + v1[...] + v2[...] + pv3[0] + pv3[1]) * 0.25)


def _prep(ego, n1, n2, pn3, s1, s2, ps3, v1, v2, pv3):
    full = pl.BlockSpec((2000, D), lambda i: (i, 0))
    part = pl.BlockSpec((2, 2000, D), lambda i: (0, i, 0))
    return pl.pallas_call(
        _prep_body,
        grid=(5,),
        in_specs=[full, full, full, part, full, full, part, full, full, part],
        out_specs=[full, full, full],
        out_shape=[jax.ShapeDtypeStruct((N_NODES, D), jnp.float32)] * 3,
    )(ego, n1, n2, pn3, s1, s2, ps3, v1, v2, pv3)


def _b1_body(g7, u1n_ref, i1n_ref, sup_ref, pos_ref):
    u = g7[0]
    i = g7[1]
    ni = g7[2]
    u1n = _normalize(g7[3])
    i1n = _normalize(g7[4])
    us = g7[5]
    iss = g7[6]
    u1n_ref[...] = u1n
    i1n_ref[...] = i1n
    sup_ref[...] = (jnp.sum(u * i, axis=1) - jnp.sum(u * ni, axis=1))[:, None]
    pos_u = jnp.sum(u1n * u, axis=1)
    pos_i = jnp.sum(i1n * i, axis=1)
    spos_u = jnp.sum(u * us, axis=1)
    spos_i = jnp.sum(i * iss, axis=1)
    pos_ref[...] = jnp.stack([pos_u, pos_i, spos_u, spos_i], axis=1)


def _b1(g7):
    return pl.pallas_call(
        _b1_body,
        out_shape=[
            jax.ShapeDtypeStruct((B, D), jnp.float32),
            jax.ShapeDtypeStruct((B, D), jnp.float32),
            jax.ShapeDtypeStruct((B, 1), jnp.float32),
            jax.ShapeDtypeStruct((B, 4), jnp.float32),
        ],
    )(g7)


def _mm_body(a_ref, b_ref, pos_ref, o_ref):
    o_ref[0] = lax.dot_general(
        a_ref[0], b_ref[0], (((1,), (1,)), ((), ())),
        preferred_element_type=jnp.float32) - pos_ref[0]


def _mm(a_stack, b_stack, pos3):
    mt = 4
    mblk = B // mt
    return pl.pallas_call(
        _mm_body,
        grid=(4, mt),
        in_specs=[
            pl.BlockSpec((1, mblk, D), lambda a, m: (a, m, 0)),
            pl.BlockSpec((1, NUM_USERS, D), lambda a, m: (a, 0, 0)),
            pl.BlockSpec((1, mblk, 1), lambda a, m: (a, m, 0)),
        ],
        out_specs=pl.BlockSpec((1, mblk, NUM_USERS), lambda a, m: (a, m, 0)),
        out_shape=jax.ShapeDtypeStruct((4, B, NUM_USERS), jnp.float32),
    )(a_stack, b_stack, pos3)


def _edge_args(idx, val):
    return idx[1], idx[0], val


def _chain(ego, idx, val, zeros):
    cols, rows, vals = _edge_args(idx, val)
    p1 = _spmm_call(ego, cols, rows, vals, zeros)
    x1 = _combine(p1)
    p2 = _spmm_call(x1, cols, rows, vals, zeros)
    x2 = _combine(p2)
    p3 = _spmm_call(x2, cols, rows, vals, zeros)
    return x1, x2, p3


def kernel(user_embeddings, item_embeddings, norm_adj_idx, norm_adj_val,
           svd_adj_idx, svd_adj_val, sub_graph1_idx, sub_graph1_val,
           sub_graph2_idx, sub_graph2_val, users, items, neg_items):
    ego = jnp.concatenate([user_embeddings, item_embeddings], axis=0)
    zeros = jnp.zeros((ROWS_PER_TILE, D), jnp.float32)

    n1, n2, pn3 = _chain(ego, norm_adj_idx, norm_adj_val, zeros)
    s1, s2, ps3 = _chain(ego, sub_graph1_idx, sub_graph1_val, zeros)
    v1, v2, pv3 = _chain(ego, svd_adj_idx, svd_adj_val, zeros)

    mn, msub1, msvd = _prep(ego, n1, n2, pn3, s1, s2, ps3, v1, v2, pv3)
    mn_lo, mn_hi = mn[:NUM_USERS], mn[NUM_USERS:]
    msvd_lo, msvd_hi = msvd[:NUM_USERS], msvd[NUM_USERS:]

    idx7 = jnp.stack([
        users, items, neg_items, users, items + NUM_USERS, users, items,
    ]).reshape(7, NW, B // NW).astype(jnp.int32)
    g7 = _gather_call(mn_lo, mn_hi, msub1, msvd_lo, msvd_hi, idx7)

    u1n, i1n, sup_col, pos_cols = _b1(g7)
    a_stack = jnp.stack([u1n, i1n, g7[0], g7[1]])
    b_stack = jnp.stack([mn_lo, mn_hi, msvd_lo, msvd_hi])
    pos3 = pos_cols.T.reshape(4, B, 1)
    out4 = _mm(a_stack, b_stack, pos3)

    return (sup_col[:, 0], out4[0], out4[1], out4[2], out4[3])
